# Initial kernel scaffold; baseline (speedup 1.0000x reference)
import jax
import jax.numpy as jnp
from jax.experimental import pallas as pl


def kernel(rna_node_features, rna_batch_idx, rna_dot_bracket_codes, W1, a_src1, a_dst1, b1, W2, a_src2, a_dst2, b2):
    raise NotImplementedError("write your pallas kernel here")


# speedup vs baseline: 10.9853x; 10.9853x over previous
import jax
import jax.numpy as jnp
from jax.experimental import pallas as pl
from functools import partial


def _leaky(x):
    return jnp.where(x >= 0, x, 0.2 * x)


def _elu(x):
    return jnp.where(x > 0, x, jnp.exp(jnp.minimum(x, 0.0)) - 1.0)


def _pair_attn(a_self_src, a_self_dst, a_part_src):
    e_self = _leaky(a_self_src + a_self_dst)
    e_part = _leaky(a_part_src + a_self_dst)
    m = jnp.maximum(e_self, e_part)
    ex_s = jnp.exp(e_self - m)
    ex_p = jnp.exp(e_part - m)
    s = ex_s + ex_p + 1e-16
    return ex_s / s, ex_p / s


def _fused_kernel(x_ref, codes_ref, w1_ref, as1_ref, ad1_ref, b1_ref,
                  w2_ref, as2_ref, ad2_ref, b2_ref, e4_ref, out_ref):
    x = x_ref[...]
    codes = codes_ref[...]
    stem_m = (codes != 0).astype(jnp.float32)
    stem_cnt = jnp.maximum(jnp.sum(stem_m, axis=1, keepdims=True), 1.0)
    loop_cnt = jnp.maximum(codes.shape[1] - jnp.sum(stem_m, axis=1, keepdims=True), 1.0)
    stem_sum = jnp.sum(x * stem_m[:, :, None], axis=1)
    loop_sum = jnp.sum(x, axis=1) - stem_sum
    S = stem_sum / stem_cnt
    Lp = loop_sum / loop_cnt

    w1 = w1_ref[...]
    a_s1 = as1_ref[...]
    a_d1 = ad1_ref[...]
    e4 = e4_ref[...]
    mm = partial(jnp.dot, preferred_element_type=jnp.float32)

    hS = mm(S, w1)
    hL = mm(Lp, w1)
    asS = mm(hS, a_s1)
    adS = mm(hS, a_d1)
    asL = mm(hL, a_s1)
    adL = mm(hL, a_d1)
    aS_self, aS_part = _pair_attn(asS, adS, asL)
    aL_self, aL_part = _pair_attn(asL, adL, asS)
    b1 = b1_ref[...]
    outS = _elu(mm(aS_self, e4) * hS + mm(aS_part, e4) * hL + b1)
    outL = _elu(mm(aL_self, e4) * hL + mm(aL_part, e4) * hS + b1)

    w2 = w2_ref[...]
    a_s2 = as2_ref[...]
    a_d2 = ad2_ref[...]
    h2S = mm(outS, w2)
    h2L = mm(outL, w2)
    as2S = mm(h2S, a_s2)
    ad2S = mm(h2S, a_d2)
    as2L = mm(h2L, a_s2)
    ad2L = mm(h2L, a_d2)
    aS2_self, aS2_part = _pair_attn(as2S, ad2S, as2L)
    aL2_self, aL2_part = _pair_attn(as2L, ad2L, as2S)
    b2 = b2_ref[...]
    out_ref[:, 0, :] = aS2_self * h2S + aS2_part * h2L + b2
    out_ref[:, 1, :] = aL2_self * h2L + aL2_part * h2S + b2


def kernel(rna_node_features, rna_batch_idx, rna_dot_bracket_codes,
           W1, a_src1, a_dst1, b1, W2, a_src2, a_dst2, b2):
    B, L = rna_dot_bracket_codes.shape
    D = rna_node_features.shape[1]
    heads, out1 = a_src1.shape
    x3 = rna_node_features.reshape(B, L, D)

    eyeh = jnp.eye(heads, dtype=jnp.float32)
    A_s1 = (eyeh[:, None, :] * a_src1[:, :, None]).reshape(heads * out1, heads)
    A_d1 = (eyeh[:, None, :] * a_dst1[:, :, None]).reshape(heads * out1, heads)
    E4 = jnp.repeat(eyeh, out1, axis=1)

    Bblk = 200
    grid = (B // Bblk,)

    out = pl.pallas_call(
        _fused_kernel,
        grid=grid,
        in_specs=[
            pl.BlockSpec((Bblk, L, D), lambda i: (i, 0, 0)),
            pl.BlockSpec((Bblk, L), lambda i: (i, 0)),
            pl.BlockSpec((D, D), lambda i: (0, 0)),
            pl.BlockSpec((D, heads), lambda i: (0, 0)),
            pl.BlockSpec((D, heads), lambda i: (0, 0)),
            pl.BlockSpec((1, D), lambda i: (0, 0)),
            pl.BlockSpec((D, D), lambda i: (0, 0)),
            pl.BlockSpec((D, 1), lambda i: (0, 0)),
            pl.BlockSpec((D, 1), lambda i: (0, 0)),
            pl.BlockSpec((1, D), lambda i: (0, 0)),
            pl.BlockSpec((heads, D), lambda i: (0, 0)),
        ],
        out_specs=pl.BlockSpec((Bblk, 2, D), lambda i: (i, 0, 0)),
        out_shape=jax.ShapeDtypeStruct((B, 2, D), jnp.float32),
    )(x3, rna_dot_bracket_codes, W1, A_s1, A_d1, b1.reshape(1, D),
      W2, a_src2.reshape(D, 1), a_dst2.reshape(D, 1), b2.reshape(1, D), E4)

    motif_batch_idx = jnp.repeat(jnp.arange(B), 2)
    return (out.reshape(2 * B, D), motif_batch_idx)
